# baseline (speedup 1.0000x reference)
import functools

import jax
import jax.numpy as jnp
from jax import lax
from jax.experimental import pallas as pl
from jax.experimental.pallas import tpu as pltpu
from jax.experimental.pallas import tpu_sc as plsc

B = 16384
D = 64
H = 2 * D

_NC = 2
_NS = 16
_NW = _NC * _NS
_BPW = B // _NW
_UNROLL = 16


def _sc_gather_both(user_table, item_table, user_ids, item_ids):
    mesh = plsc.VectorSubcoreMesh(core_axis_name="c", subcore_axis_name="s")

    @functools.partial(
        pl.kernel,
        mesh=mesh,
        out_type=(
            jax.ShapeDtypeStruct((B, D), jnp.float32),
            jax.ShapeDtypeStruct((B, D), jnp.float32),
        ),
        scratch_types=[
            pltpu.VMEM((_BPW,), jnp.int32),
            pltpu.VMEM((_BPW,), jnp.int32),
            pltpu.SemaphoreType.DMA,
            pltpu.SemaphoreType.DMA,
        ],
    )
    def k(ut_hbm, it_hbm, uid_hbm, iid_hbm, uout_hbm, iout_hbm,
          uid_s, iid_s, sem_u, sem_i):
        wid = lax.axis_index("s") * _NC + lax.axis_index("c")
        base = wid * _BPW
        sl = pl.ds(base, _BPW)
        pltpu.sync_copy(uid_hbm.at[sl], uid_s)
        pltpu.sync_copy(iid_hbm.at[sl], iid_s)

        @pl.loop(0, _BPW, step=_UNROLL)
        def _(i):
            uv = uid_s[pl.ds(i, _UNROLL)]
            iv = iid_s[pl.ds(i, _UNROLL)]
            for j in range(_UNROLL):
                pltpu.async_copy(ut_hbm.at[pl.ds(uv[j], 1)],
                                 uout_hbm.at[pl.ds(base + i + j, 1)], sem_u)
                pltpu.async_copy(it_hbm.at[pl.ds(iv[j], 1)],
                                 iout_hbm.at[pl.ds(base + i + j, 1)], sem_i)

        pltpu.make_async_copy(ut_hbm.at[pl.ds(0, _BPW)],
                              uout_hbm.at[sl], sem_u).wait()
        pltpu.make_async_copy(it_hbm.at[pl.ds(0, _BPW)],
                              iout_hbm.at[sl], sem_i).wait()

    return k(user_table, item_table, user_ids, item_ids)


_BLK = 2048


def _mlp_body(eu_ref, ei_ref,
              uw1, ub1, uw2, ub2, iw1, ib1, iw2, ib2,
              ou_ref, oi_ref):
    def tower(e, w1, b1, w2, b2):
        h = jnp.dot(e, w1, preferred_element_type=jnp.float32,
                    precision=lax.Precision.HIGHEST)
        h = jnp.maximum(h + b1, 0.0)
        o = jnp.dot(h, w2, preferred_element_type=jnp.float32,
                    precision=lax.Precision.HIGHEST)
        o = o + b2
        norm = jnp.sqrt(jnp.sum(o * o, axis=1, keepdims=True))
        return o / jnp.maximum(norm, 1e-12)

    ou_ref[...] = tower(eu_ref[...], uw1[...], ub1[...], uw2[...], ub2[...])
    oi_ref[...] = tower(ei_ref[...], iw1[...], ib1[...], iw2[...], ib2[...])


def _mlp_norm(e_u, e_i, uW1, ub1, uW2, ub2, iW1, ib1, iW2, ib2):
    blk = pl.BlockSpec((_BLK, D), lambda i: (i, 0))
    full = lambda shape: pl.BlockSpec(shape, lambda i: tuple(0 for _ in shape))
    return pl.pallas_call(
        _mlp_body,
        grid=(B // _BLK,),
        in_specs=[
            blk, blk,
            full((D, H)), full((1, H)), full((H, D)), full((1, D)),
            full((D, H)), full((1, H)), full((H, D)), full((1, D)),
        ],
        out_specs=[blk, blk],
        out_shape=(
            jax.ShapeDtypeStruct((B, D), jnp.float32),
            jax.ShapeDtypeStruct((B, D), jnp.float32),
        ),
    )(e_u, e_i, uW1, ub1, uW2, ub2, iW1, ib1, iW2, ib2)


def kernel(user_ids, item_ids, user_table, item_table,
           uW1, ub1, uW2, ub2, iW1, ib1, iW2, ib2):
    e_u, e_i = _sc_gather_both(user_table, item_table, user_ids, item_ids)
    u_vec, i_vec = _mlp_norm(
        e_u, e_i,
        uW1, ub1.reshape(1, H), uW2, ub2.reshape(1, D),
        iW1, ib1.reshape(1, H), iW2, ib2.reshape(1, D),
    )
    return (u_vec, i_vec)


# speedup vs baseline: 1.7789x; 1.7789x over previous
import functools

import jax
import jax.numpy as jnp
from jax import lax
from jax.experimental import pallas as pl
from jax.experimental.pallas import tpu as pltpu
from jax.experimental.pallas import tpu_sc as plsc

B = 16384
D = 64
H = 2 * D
V = 1000000

_NC = 2
_NS = 16
_NW = _NC * _NS
_BPW = B // _NW
_CH = 256

_PB = 4096


def _prep_body(ut_ref, it_ref, ey_ref, gu_ref, gi_ref):
    ey = ey_ref[...]

    def tr(x):
        return lax.dot_general(x, ey, (((0,), (0,)), ((), ())),
                               preferred_element_type=jnp.float32)

    xu = tr(ut_ref[...])
    gu_ref[...] = jnp.concatenate([xu, xu], axis=1)
    xi = tr(it_ref[...])
    gi_ref[...] = jnp.concatenate([xi, xi], axis=1)


def _prep(user_table_t, item_table_t, eye):
    inb = pl.BlockSpec((D, _PB), lambda i: (0, i))
    outb = pl.BlockSpec((_PB, 2 * D), lambda i: (i, 0))
    return pl.pallas_call(
        _prep_body,
        grid=(pl.cdiv(V, _PB),),
        in_specs=[inb, inb, pl.BlockSpec((D, D), lambda i: (0, 0))],
        out_specs=[outb, outb],
        out_shape=(
            jax.ShapeDtypeStruct((V, 2 * D), jnp.float32),
            jax.ShapeDtypeStruct((V, 2 * D), jnp.float32),
        ),
        compiler_params=pltpu.CompilerParams(
            dimension_semantics=("parallel",)),
    )(user_table_t, item_table_t, eye)


def _sc_gather_both(gu, gi, user_ids, item_ids):
    mesh = plsc.VectorSubcoreMesh(core_axis_name="c", subcore_axis_name="s")

    @functools.partial(
        pl.kernel,
        mesh=mesh,
        out_type=(
            jax.ShapeDtypeStruct((B, 2 * D), jnp.float32),
            jax.ShapeDtypeStruct((B, 2 * D), jnp.float32),
        ),
        scratch_types=[
            pltpu.VMEM((_BPW,), jnp.int32),
            pltpu.VMEM((_BPW,), jnp.int32),
            pltpu.VMEM((_CH, 2 * D), jnp.float32),
            pltpu.VMEM((_CH, 2 * D), jnp.float32),
            pltpu.SemaphoreType.DMA,
            pltpu.SemaphoreType.DMA,
            pltpu.SemaphoreType.DMA,
            pltpu.SemaphoreType.DMA,
        ],
    )
    def k(ut_hbm, it_hbm, uid_hbm, iid_hbm, uout_hbm, iout_hbm,
          uidx_v, iidx_v, rows_a, rows_b, sem_a, sem_b, sem_sa, sem_sb):
        wid = lax.axis_index("s") * _NC + lax.axis_index("c")
        base = wid * _BPW
        sl = pl.ds(base, _BPW)
        pltpu.sync_copy(uid_hbm.at[sl], uidx_v)
        pltpu.sync_copy(iid_hbm.at[sl], iidx_v)

        work = []
        for tbl, idx_v, out_hbm in ((ut_hbm, uidx_v, uout_hbm),
                                    (it_hbm, iidx_v, iout_hbm)):
            for c in range(_BPW // _CH):
                work.append((tbl, idx_v.at[pl.ds(c * _CH, _CH)],
                             out_hbm.at[pl.ds(base + c * _CH, _CH)]))

        bufs = (rows_a, rows_b)
        gsems = (sem_a, sem_b)
        ssems = (sem_sa, sem_sb)
        n = len(work)
        gath = [None] * n
        stor = [None] * n
        for i in range(n):
            b = i % 2
            if i >= 2:
                stor[i - 2].wait()
            tbl, idx, out = work[i]
            gath[i] = pltpu.async_copy(tbl.at[idx], bufs[b], gsems[b])
            if i >= 1:
                gath[i - 1].wait()
                _, _, prev_out = work[i - 1]
                stor[i - 1] = pltpu.async_copy(bufs[(i - 1) % 2], prev_out,
                                               ssems[(i - 1) % 2])
        gath[n - 1].wait()
        stor[n - 1] = pltpu.async_copy(bufs[(n - 1) % 2], work[n - 1][2],
                                       ssems[(n - 1) % 2])
        stor[n - 2].wait()
        stor[n - 1].wait()

    return k(gu, gi, user_ids, item_ids)


_BLK = 2048


def _mlp_body(eu_ref, ei_ref,
              uw1, ub1, uw2, ub2, iw1, ib1, iw2, ib2,
              ou_ref, oi_ref):
    def tower(e2, w1, b1, w2, b2):
        e = e2[:, :D]
        h = jnp.dot(e, w1, preferred_element_type=jnp.float32)
        h = jnp.maximum(h + b1, 0.0)
        o = jnp.dot(h, w2, preferred_element_type=jnp.float32)
        o = o + b2
        norm = jnp.sqrt(jnp.sum(o * o, axis=1, keepdims=True))
        return o / jnp.maximum(norm, 1e-12)

    ou_ref[...] = tower(eu_ref[...], uw1[...], ub1[...], uw2[...], ub2[...])
    oi_ref[...] = tower(ei_ref[...], iw1[...], ib1[...], iw2[...], ib2[...])


def _mlp_norm(e2_u, e2_i, uW1, ub1, uW2, ub2, iW1, ib1, iW2, ib2):
    blk2 = pl.BlockSpec((_BLK, 2 * D), lambda i: (i, 0))
    blk = pl.BlockSpec((_BLK, D), lambda i: (i, 0))
    full = lambda shape: pl.BlockSpec(shape, lambda i: tuple(0 for _ in shape))
    return pl.pallas_call(
        _mlp_body,
        grid=(B // _BLK,),
        in_specs=[
            blk2, blk2,
            full((D, H)), full((1, H)), full((H, D)), full((1, D)),
            full((D, H)), full((1, H)), full((H, D)), full((1, D)),
        ],
        out_specs=[blk, blk],
        out_shape=(
            jax.ShapeDtypeStruct((B, D), jnp.float32),
            jax.ShapeDtypeStruct((B, D), jnp.float32),
        ),
    )(e2_u, e2_i, uW1, ub1, uW2, ub2, iW1, ib1, iW2, ib2)


def kernel(user_ids, item_ids, user_table, item_table,
           uW1, ub1, uW2, ub2, iW1, ib1, iW2, ib2):
    eye = jnp.eye(D, dtype=jnp.float32)
    gu, gi = _prep(user_table.T, item_table.T, eye)
    e2_u, e2_i = _sc_gather_both(gu, gi, user_ids, item_ids)
    u_vec, i_vec = _mlp_norm(
        e2_u, e2_i,
        uW1, ub1.reshape(1, H), uW2, ub2.reshape(1, D),
        iW1, ib1.reshape(1, H), iW2, ib2.reshape(1, D),
    )
    return (u_vec, i_vec)


# speedup vs baseline: 2.7317x; 1.5356x over previous
import functools

import jax
import jax.numpy as jnp
from jax import lax
from jax.experimental import pallas as pl
from jax.experimental.pallas import tpu as pltpu
from jax.experimental.pallas import tpu_sc as plsc

B = 16384
D = 64
H = 2 * D
V = 1000000

_NC = 2
_NS = 16
_NW = _NC * _NS
_BPW = B // _NW
_CH = 256

_PB = 4096
_NPB = -(-V // _PB)
_GV = _NPB * (_PB // 2)


def _prep_body(ut_ref, it_ref, ey_ref, gu_ref, gi_ref):
    ey = ey_ref[...]

    def tr(x):
        xp = jnp.concatenate([x[:, :_PB // 2], x[:, _PB // 2:]], axis=0)
        return lax.dot_general(xp, ey, (((0,), (0,)), ((), ())),
                               preferred_element_type=jnp.float32)

    gu_ref[...] = tr(ut_ref[...])
    gi_ref[...] = tr(it_ref[...])


def _prep(user_table_t, item_table_t, eye):
    inb = pl.BlockSpec((D, _PB), lambda i: (0, i))
    outb = pl.BlockSpec((_PB // 2, 2 * D), lambda i: (i, 0))
    return pl.pallas_call(
        _prep_body,
        grid=(pl.cdiv(V, _PB),),
        in_specs=[inb, inb, pl.BlockSpec((2 * D, 2 * D), lambda i: (0, 0))],
        out_specs=[outb, outb],
        out_shape=(
            jax.ShapeDtypeStruct((_GV, 2 * D), jnp.float32),
            jax.ShapeDtypeStruct((_GV, 2 * D), jnp.float32),
        ),
        compiler_params=pltpu.CompilerParams(
            dimension_semantics=("parallel",)),
    )(user_table_t, item_table_t, eye)


def _sc_gather_both(gu, gi, user_ids, item_ids):
    mesh = plsc.VectorSubcoreMesh(core_axis_name="c", subcore_axis_name="s")

    @functools.partial(
        pl.kernel,
        mesh=mesh,
        out_type=(
            jax.ShapeDtypeStruct((B, 2 * D), jnp.float32),
            jax.ShapeDtypeStruct((B, 2 * D), jnp.float32),
        ),
        scratch_types=[
            pltpu.VMEM((_BPW,), jnp.int32),
            pltpu.VMEM((_BPW,), jnp.int32),
            pltpu.VMEM((_CH, 2 * D), jnp.float32),
            pltpu.VMEM((_CH, 2 * D), jnp.float32),
            pltpu.SemaphoreType.DMA,
            pltpu.SemaphoreType.DMA,
            pltpu.SemaphoreType.DMA,
            pltpu.SemaphoreType.DMA,
        ],
    )
    def k(ut_hbm, it_hbm, uid_hbm, iid_hbm, uout_hbm, iout_hbm,
          uidx_v, iidx_v, rows_a, rows_b, sem_a, sem_b, sem_sa, sem_sb):
        wid = lax.axis_index("s") * _NC + lax.axis_index("c")
        base = wid * _BPW
        sl = pl.ds(base, _BPW)
        pltpu.sync_copy(uid_hbm.at[sl], uidx_v)
        pltpu.sync_copy(iid_hbm.at[sl], iidx_v)

        @pl.loop(0, _BPW, step=16)
        def _(c):
            s = pl.ds(c, 16)
            u = uidx_v.at[s][...]
            uidx_v.at[s][...] = (
                lax.shift_left(lax.shift_right_logical(u, 12), 11)
                + jnp.bitwise_and(u, 2047))
            v = iidx_v.at[s][...]
            iidx_v.at[s][...] = (
                lax.shift_left(lax.shift_right_logical(v, 12), 11)
                + jnp.bitwise_and(v, 2047))

        work = []
        for tbl, idx_v, out_hbm in ((ut_hbm, uidx_v, uout_hbm),
                                    (it_hbm, iidx_v, iout_hbm)):
            for c in range(_BPW // _CH):
                work.append((tbl, idx_v.at[pl.ds(c * _CH, _CH)],
                             out_hbm.at[pl.ds(base + c * _CH, _CH)]))

        bufs = (rows_a, rows_b)
        gsems = (sem_a, sem_b)
        ssems = (sem_sa, sem_sb)
        n = len(work)
        gath = [None] * n
        stor = [None] * n
        for i in range(n):
            b = i % 2
            if i >= 2:
                stor[i - 2].wait()
            tbl, idx, out = work[i]
            gath[i] = pltpu.async_copy(tbl.at[idx], bufs[b], gsems[b])
            if i >= 1:
                gath[i - 1].wait()
                _, _, prev_out = work[i - 1]
                stor[i - 1] = pltpu.async_copy(bufs[(i - 1) % 2], prev_out,
                                               ssems[(i - 1) % 2])
        gath[n - 1].wait()
        stor[n - 1] = pltpu.async_copy(bufs[(n - 1) % 2], work[n - 1][2],
                                       ssems[(n - 1) % 2])
        stor[n - 2].wait()
        stor[n - 1].wait()

    return k(gu, gi, user_ids, item_ids)


_BLK = 2048


def _mlp_body(eu_ref, ei_ref, uids_ref, iids_ref,
              uw1, ub1, uw2, ub2, iw1, ib1, iw2, ib2,
              ou_ref, oi_ref):
    def tower(e2, ids, w1, b1, w2, b2):
        hi = jnp.bitwise_and(ids, 2048) == 2048
        e = jnp.where(hi, e2[:, D:], e2[:, :D])
        h = jnp.dot(e, w1, preferred_element_type=jnp.float32)
        h = jnp.maximum(h + b1, 0.0)
        o = jnp.dot(h, w2, preferred_element_type=jnp.float32)
        o = o + b2
        norm = jnp.sqrt(jnp.sum(o * o, axis=1, keepdims=True))
        return o / jnp.maximum(norm, 1e-12)

    ou_ref[...] = tower(eu_ref[...], uids_ref[...],
                        uw1[...], ub1[...], uw2[...], ub2[...])
    oi_ref[...] = tower(ei_ref[...], iids_ref[...],
                        iw1[...], ib1[...], iw2[...], ib2[...])


def _mlp_norm(e2_u, e2_i, uids, iids, uW1, ub1, uW2, ub2, iW1, ib1, iW2, ib2):
    blk2 = pl.BlockSpec((_BLK, 2 * D), lambda i: (i, 0))
    blk = pl.BlockSpec((_BLK, D), lambda i: (i, 0))
    ids_spec = pl.BlockSpec((_BLK, 1), lambda i: (i, 0))
    full = lambda shape: pl.BlockSpec(shape, lambda i: tuple(0 for _ in shape))
    return pl.pallas_call(
        _mlp_body,
        grid=(B // _BLK,),
        in_specs=[
            blk2, blk2, ids_spec, ids_spec,
            full((D, H)), full((1, H)), full((H, D)), full((1, D)),
            full((D, H)), full((1, H)), full((H, D)), full((1, D)),
        ],
        out_specs=[blk, blk],
        out_shape=(
            jax.ShapeDtypeStruct((B, D), jnp.float32),
            jax.ShapeDtypeStruct((B, D), jnp.float32),
        ),
    )(e2_u, e2_i, uids, iids, uW1, ub1, uW2, ub2, iW1, ib1, iW2, ib2)


def kernel(user_ids, item_ids, user_table, item_table,
           uW1, ub1, uW2, ub2, iW1, ib1, iW2, ib2):
    eye = jnp.eye(2 * D, dtype=jnp.float32)
    gu, gi = _prep(user_table.T, item_table.T, eye)
    e2_u, e2_i = _sc_gather_both(gu, gi, user_ids, item_ids)
    u_vec, i_vec = _mlp_norm(
        e2_u, e2_i,
        user_ids.reshape(B, 1), item_ids.reshape(B, 1),
        uW1, ub1.reshape(1, H), uW2, ub2.reshape(1, D),
        iW1, ib1.reshape(1, H), iW2, ib2.reshape(1, D),
    )
    return (u_vec, i_vec)


# speedup vs baseline: 3.1261x; 1.1444x over previous
import functools

import jax
import jax.numpy as jnp
from jax import lax
from jax.experimental import pallas as pl
from jax.experimental.pallas import tpu as pltpu
from jax.experimental.pallas import tpu_sc as plsc

B = 16384
D = 64
H = 2 * D
V = 1000000

_NC = 2
_NS = 16
_NW = _NC * _NS
_BPW = B // _NW
_CH = 256

_PB = 8192
_HPB = _PB // 2
_PSH = _PB.bit_length() - 1
_NPB = -(-V // _PB)
_GV = _NPB * _HPB


def _prep_body(ut_ref, it_ref, ey_ref, gu_ref, gi_ref):
    ey = ey_ref[...]

    def tr(x):
        xp = jnp.concatenate([x[:, :_HPB], x[:, _HPB:]], axis=0)
        return lax.dot_general(xp, ey, (((0,), (0,)), ((), ())),
                               preferred_element_type=jnp.float32)

    gu_ref[...] = tr(ut_ref[...])
    gi_ref[...] = tr(it_ref[...])


def _prep(user_table_t, item_table_t, eye):
    inb = pl.BlockSpec((D, _PB), lambda i: (0, i))
    outb = pl.BlockSpec((_HPB, 2 * D), lambda i: (i, 0))
    return pl.pallas_call(
        _prep_body,
        grid=(pl.cdiv(V, _PB),),
        in_specs=[inb, inb, pl.BlockSpec((2 * D, 2 * D), lambda i: (0, 0))],
        out_specs=[outb, outb],
        out_shape=(
            jax.ShapeDtypeStruct((_GV, 2 * D), jnp.float32),
            jax.ShapeDtypeStruct((_GV, 2 * D), jnp.float32),
        ),
        compiler_params=pltpu.CompilerParams(
            dimension_semantics=("parallel",)),
    )(user_table_t, item_table_t, eye)


def _sc_gather_both(gu, gi, user_ids, item_ids):
    mesh = plsc.VectorSubcoreMesh(core_axis_name="c", subcore_axis_name="s")

    @functools.partial(
        pl.kernel,
        mesh=mesh,
        out_type=(
            jax.ShapeDtypeStruct((B, 2 * D), jnp.float32),
            jax.ShapeDtypeStruct((B, 2 * D), jnp.float32),
        ),
        scratch_types=[
            pltpu.VMEM((_BPW,), jnp.int32),
            pltpu.VMEM((_BPW,), jnp.int32),
            pltpu.VMEM((_CH, 2 * D), jnp.float32),
            pltpu.VMEM((_CH, 2 * D), jnp.float32),
            pltpu.SemaphoreType.DMA,
            pltpu.SemaphoreType.DMA,
            pltpu.SemaphoreType.DMA,
            pltpu.SemaphoreType.DMA,
        ],
    )
    def k(ut_hbm, it_hbm, uid_hbm, iid_hbm, uout_hbm, iout_hbm,
          uidx_v, iidx_v, rows_a, rows_b, sem_a, sem_b, sem_sa, sem_sb):
        wid = lax.axis_index("s") * _NC + lax.axis_index("c")
        base = wid * _BPW
        sl = pl.ds(base, _BPW)
        pltpu.sync_copy(uid_hbm.at[sl], uidx_v)
        pltpu.sync_copy(iid_hbm.at[sl], iidx_v)

        @pl.loop(0, _BPW, step=16)
        def _(c):
            s = pl.ds(c, 16)
            u = uidx_v.at[s][...]
            uidx_v.at[s][...] = (
                lax.shift_left(lax.shift_right_logical(u, _PSH), _PSH - 1)
                + jnp.bitwise_and(u, _HPB - 1))
            v = iidx_v.at[s][...]
            iidx_v.at[s][...] = (
                lax.shift_left(lax.shift_right_logical(v, _PSH), _PSH - 1)
                + jnp.bitwise_and(v, _HPB - 1))

        work = []
        for tbl, idx_v, out_hbm in ((ut_hbm, uidx_v, uout_hbm),
                                    (it_hbm, iidx_v, iout_hbm)):
            for c in range(_BPW // _CH):
                work.append((tbl, idx_v.at[pl.ds(c * _CH, _CH)],
                             out_hbm.at[pl.ds(base + c * _CH, _CH)]))

        bufs = (rows_a, rows_b)
        gsems = (sem_a, sem_b)
        ssems = (sem_sa, sem_sb)
        n = len(work)
        gath = [None] * n
        stor = [None] * n
        for i in range(n):
            b = i % 2
            if i >= 2:
                stor[i - 2].wait()
            tbl, idx, out = work[i]
            gath[i] = pltpu.async_copy(tbl.at[idx], bufs[b], gsems[b])
            if i >= 1:
                gath[i - 1].wait()
                _, _, prev_out = work[i - 1]
                stor[i - 1] = pltpu.async_copy(bufs[(i - 1) % 2], prev_out,
                                               ssems[(i - 1) % 2])
        gath[n - 1].wait()
        stor[n - 1] = pltpu.async_copy(bufs[(n - 1) % 2], work[n - 1][2],
                                       ssems[(n - 1) % 2])
        stor[n - 2].wait()
        stor[n - 1].wait()

    return k(gu, gi, user_ids, item_ids)


_BLK = 2048


def _mlp_body(eu_ref, ei_ref, uids_ref, iids_ref,
              uw1, ub1, uw2, ub2, iw1, ib1, iw2, ib2,
              ou_ref, oi_ref):
    def tower(e2, ids, w1, b1, w2, b2):
        hi = jnp.bitwise_and(ids, _HPB) == _HPB
        e = jnp.where(hi, e2[:, D:], e2[:, :D])
        h = jnp.dot(e, w1, preferred_element_type=jnp.float32)
        h = jnp.maximum(h + b1, 0.0)
        o = jnp.dot(h, w2, preferred_element_type=jnp.float32)
        o = o + b2
        norm = jnp.sqrt(jnp.sum(o * o, axis=1, keepdims=True))
        return o / jnp.maximum(norm, 1e-12)

    ou_ref[...] = tower(eu_ref[...], uids_ref[...],
                        uw1[...], ub1[...], uw2[...], ub2[...])
    oi_ref[...] = tower(ei_ref[...], iids_ref[...],
                        iw1[...], ib1[...], iw2[...], ib2[...])


def _mlp_norm(e2_u, e2_i, uids, iids, uW1, ub1, uW2, ub2, iW1, ib1, iW2, ib2):
    blk2 = pl.BlockSpec((_BLK, 2 * D), lambda i: (i, 0))
    blk = pl.BlockSpec((_BLK, D), lambda i: (i, 0))
    ids_spec = pl.BlockSpec((_BLK, 1), lambda i: (i, 0))
    full = lambda shape: pl.BlockSpec(shape, lambda i: tuple(0 for _ in shape))
    return pl.pallas_call(
        _mlp_body,
        grid=(B // _BLK,),
        in_specs=[
            blk2, blk2, ids_spec, ids_spec,
            full((D, H)), full((1, H)), full((H, D)), full((1, D)),
            full((D, H)), full((1, H)), full((H, D)), full((1, D)),
        ],
        out_specs=[blk, blk],
        out_shape=(
            jax.ShapeDtypeStruct((B, D), jnp.float32),
            jax.ShapeDtypeStruct((B, D), jnp.float32),
        ),
    )(e2_u, e2_i, uids, iids, uW1, ub1, uW2, ub2, iW1, ib1, iW2, ib2)


def kernel(user_ids, item_ids, user_table, item_table,
           uW1, ub1, uW2, ub2, iW1, ib1, iW2, ib2):
    eye = jnp.eye(2 * D, dtype=jnp.float32)
    gu, gi = _prep(user_table.T, item_table.T, eye)
    e2_u, e2_i = _sc_gather_both(gu, gi, user_ids, item_ids)
    u_vec, i_vec = _mlp_norm(
        e2_u, e2_i,
        user_ids.reshape(B, 1), item_ids.reshape(B, 1),
        uW1, ub1.reshape(1, H), uW2, ub2.reshape(1, D),
        iW1, ib1.reshape(1, H), iW2, ib2.reshape(1, D),
    )
    return (u_vec, i_vec)


# speedup vs baseline: 3.1722x; 1.0147x over previous
import functools

import jax
import jax.numpy as jnp
from jax import lax
from jax.experimental import pallas as pl
from jax.experimental.pallas import tpu as pltpu
from jax.experimental.pallas import tpu_sc as plsc

B = 16384
D = 64
H = 2 * D
V = 1000000

_NC = 2
_NS = 16
_NW = _NC * _NS
_BPW = B // _NW
_CH = 256

_PB = 16384
_HPB = _PB // 2
_PSH = _PB.bit_length() - 1
_NPB = -(-V // _PB)
_GV = _NPB * _HPB


def _prep_body(ut_ref, it_ref, ey_ref, gu_ref, gi_ref):
    ey = ey_ref[...]

    def tr(x):
        xp = jnp.concatenate([x[:, :_HPB], x[:, _HPB:]], axis=0)
        return lax.dot_general(xp, ey, (((0,), (0,)), ((), ())),
                               preferred_element_type=jnp.float32)

    gu_ref[...] = tr(ut_ref[...])
    gi_ref[...] = tr(it_ref[...])


def _prep(user_table_t, item_table_t, eye):
    inb = pl.BlockSpec((D, _PB), lambda i: (0, i))
    outb = pl.BlockSpec((_HPB, 2 * D), lambda i: (i, 0))
    return pl.pallas_call(
        _prep_body,
        grid=(pl.cdiv(V, _PB),),
        in_specs=[inb, inb, pl.BlockSpec((2 * D, 2 * D), lambda i: (0, 0))],
        out_specs=[outb, outb],
        out_shape=(
            jax.ShapeDtypeStruct((_GV, 2 * D), jnp.float32),
            jax.ShapeDtypeStruct((_GV, 2 * D), jnp.float32),
        ),
        compiler_params=pltpu.CompilerParams(
            dimension_semantics=("parallel",)),
    )(user_table_t, item_table_t, eye)


def _sc_gather_both(gu, gi, user_ids, item_ids):
    mesh = plsc.VectorSubcoreMesh(core_axis_name="c", subcore_axis_name="s")

    @functools.partial(
        pl.kernel,
        mesh=mesh,
        out_type=(
            jax.ShapeDtypeStruct((B, 2 * D), jnp.float32),
            jax.ShapeDtypeStruct((B, 2 * D), jnp.float32),
        ),
        scratch_types=[
            pltpu.VMEM((_BPW,), jnp.int32),
            pltpu.VMEM((_BPW,), jnp.int32),
            pltpu.VMEM((_CH, 2 * D), jnp.float32),
            pltpu.VMEM((_CH, 2 * D), jnp.float32),
            pltpu.SemaphoreType.DMA,
            pltpu.SemaphoreType.DMA,
            pltpu.SemaphoreType.DMA,
            pltpu.SemaphoreType.DMA,
        ],
    )
    def k(ut_hbm, it_hbm, uid_hbm, iid_hbm, uout_hbm, iout_hbm,
          uidx_v, iidx_v, rows_a, rows_b, sem_a, sem_b, sem_sa, sem_sb):
        wid = lax.axis_index("s") * _NC + lax.axis_index("c")
        base = wid * _BPW
        sl = pl.ds(base, _BPW)
        pltpu.sync_copy(uid_hbm.at[sl], uidx_v)
        pltpu.sync_copy(iid_hbm.at[sl], iidx_v)

        @pl.loop(0, _BPW, step=16)
        def _(c):
            s = pl.ds(c, 16)
            u = uidx_v.at[s][...]
            uidx_v.at[s][...] = (
                lax.shift_left(lax.shift_right_logical(u, _PSH), _PSH - 1)
                + jnp.bitwise_and(u, _HPB - 1))
            v = iidx_v.at[s][...]
            iidx_v.at[s][...] = (
                lax.shift_left(lax.shift_right_logical(v, _PSH), _PSH - 1)
                + jnp.bitwise_and(v, _HPB - 1))

        work = []
        for tbl, idx_v, out_hbm in ((ut_hbm, uidx_v, uout_hbm),
                                    (it_hbm, iidx_v, iout_hbm)):
            for c in range(_BPW // _CH):
                work.append((tbl, idx_v.at[pl.ds(c * _CH, _CH)],
                             out_hbm.at[pl.ds(base + c * _CH, _CH)]))

        bufs = (rows_a, rows_b)
        gsems = (sem_a, sem_b)
        ssems = (sem_sa, sem_sb)
        n = len(work)
        gath = [None] * n
        stor = [None] * n
        for i in range(n):
            b = i % 2
            if i >= 2:
                stor[i - 2].wait()
            tbl, idx, out = work[i]
            gath[i] = pltpu.async_copy(tbl.at[idx], bufs[b], gsems[b])
            if i >= 1:
                gath[i - 1].wait()
                _, _, prev_out = work[i - 1]
                stor[i - 1] = pltpu.async_copy(bufs[(i - 1) % 2], prev_out,
                                               ssems[(i - 1) % 2])
        gath[n - 1].wait()
        stor[n - 1] = pltpu.async_copy(bufs[(n - 1) % 2], work[n - 1][2],
                                       ssems[(n - 1) % 2])
        stor[n - 2].wait()
        stor[n - 1].wait()

    return k(gu, gi, user_ids, item_ids)


_BLK = 2048


def _mlp_body(eu_ref, ei_ref, uids_ref, iids_ref,
              uw1, ub1, uw2, ub2, iw1, ib1, iw2, ib2,
              ou_ref, oi_ref):
    def tower(e2, ids, w1, b1, w2, b2):
        hi = jnp.bitwise_and(ids, _HPB) == _HPB
        e = jnp.where(hi, e2[:, D:], e2[:, :D])
        h = jnp.dot(e, w1, preferred_element_type=jnp.float32)
        h = jnp.maximum(h + b1, 0.0)
        o = jnp.dot(h, w2, preferred_element_type=jnp.float32)
        o = o + b2
        norm = jnp.sqrt(jnp.sum(o * o, axis=1, keepdims=True))
        return o / jnp.maximum(norm, 1e-12)

    ou_ref[...] = tower(eu_ref[...], uids_ref[...],
                        uw1[...], ub1[...], uw2[...], ub2[...])
    oi_ref[...] = tower(ei_ref[...], iids_ref[...],
                        iw1[...], ib1[...], iw2[...], ib2[...])


def _mlp_norm(e2_u, e2_i, uids, iids, uW1, ub1, uW2, ub2, iW1, ib1, iW2, ib2):
    blk2 = pl.BlockSpec((_BLK, 2 * D), lambda i: (i, 0))
    blk = pl.BlockSpec((_BLK, D), lambda i: (i, 0))
    ids_spec = pl.BlockSpec((_BLK, 1), lambda i: (i, 0))
    full = lambda shape: pl.BlockSpec(shape, lambda i: tuple(0 for _ in shape))
    return pl.pallas_call(
        _mlp_body,
        grid=(B // _BLK,),
        in_specs=[
            blk2, blk2, ids_spec, ids_spec,
            full((D, H)), full((1, H)), full((H, D)), full((1, D)),
            full((D, H)), full((1, H)), full((H, D)), full((1, D)),
        ],
        out_specs=[blk, blk],
        out_shape=(
            jax.ShapeDtypeStruct((B, D), jnp.float32),
            jax.ShapeDtypeStruct((B, D), jnp.float32),
        ),
    )(e2_u, e2_i, uids, iids, uW1, ub1, uW2, ub2, iW1, ib1, iW2, ib2)


def kernel(user_ids, item_ids, user_table, item_table,
           uW1, ub1, uW2, ub2, iW1, ib1, iW2, ib2):
    eye = jnp.eye(2 * D, dtype=jnp.float32)
    gu, gi = _prep(user_table.T, item_table.T, eye)
    e2_u, e2_i = _sc_gather_both(gu, gi, user_ids, item_ids)
    u_vec, i_vec = _mlp_norm(
        e2_u, e2_i,
        user_ids.reshape(B, 1), item_ids.reshape(B, 1),
        uW1, ub1.reshape(1, H), uW2, ub2.reshape(1, D),
        iW1, ib1.reshape(1, H), iW2, ib2.reshape(1, D),
    )
    return (u_vec, i_vec)


# speedup vs baseline: 3.8637x; 1.2180x over previous
import functools

import jax
import jax.numpy as jnp
from jax import lax
from jax.experimental import pallas as pl
from jax.experimental.pallas import tpu as pltpu
from jax.experimental.pallas import tpu_sc as plsc

B = 16384
D = 64
H = 2 * D
V = 1000000

_NC = 2
_NS = 16
_NW = _NC * _NS
_BPW = B // _NW
_CH = 256

_PB = 16384
_QPB = _PB // 4
_PSH = _PB.bit_length() - 1
_NPB = -(-V // _PB)
_GV = _NPB * _QPB


def _prep_body(ut_ref, it_ref, ey_ref, gu_ref, gi_ref):
    ey = ey_ref[...]

    def tr(x):
        xp = jnp.concatenate(
            [x[:, i * _QPB:(i + 1) * _QPB] for i in range(4)], axis=0)
        return lax.dot_general(xp, ey, (((0,), (0,)), ((), ())),
                               preferred_element_type=jnp.float32)

    def pack(xt):
        lo = xt[:, :2 * D].astype(jnp.bfloat16)
        hi = xt[:, 2 * D:].astype(jnp.bfloat16)
        lo32 = lax.bitcast_convert_type(lo, jnp.uint16).astype(jnp.uint32)
        hi32 = lax.bitcast_convert_type(hi, jnp.uint16).astype(jnp.uint32)
        w = jnp.bitwise_or(lo32, jnp.left_shift(hi32, jnp.uint32(16)))
        return lax.bitcast_convert_type(w, jnp.int32)

    gu_ref[...] = pack(tr(ut_ref[...]))
    gi_ref[...] = pack(tr(it_ref[...]))


def _prep(user_table_t, item_table_t, eye):
    inb = pl.BlockSpec((D, _PB), lambda i: (0, i))
    outb = pl.BlockSpec((_QPB, 2 * D), lambda i: (i, 0))
    return pl.pallas_call(
        _prep_body,
        grid=(pl.cdiv(V, _PB),),
        in_specs=[inb, inb, pl.BlockSpec((4 * D, 4 * D), lambda i: (0, 0))],
        out_specs=[outb, outb],
        out_shape=(
            jax.ShapeDtypeStruct((_GV, 2 * D), jnp.int32),
            jax.ShapeDtypeStruct((_GV, 2 * D), jnp.int32),
        ),
        compiler_params=pltpu.CompilerParams(
            dimension_semantics=("parallel",)),
    )(user_table_t, item_table_t, eye)


def _sc_gather_both(gu, gi, user_ids, item_ids):
    mesh = plsc.VectorSubcoreMesh(core_axis_name="c", subcore_axis_name="s")

    @functools.partial(
        pl.kernel,
        mesh=mesh,
        out_type=(
            jax.ShapeDtypeStruct((B, 2 * D), jnp.int32),
            jax.ShapeDtypeStruct((B, 2 * D), jnp.int32),
        ),
        scratch_types=[
            pltpu.VMEM((_BPW,), jnp.int32),
            pltpu.VMEM((_BPW,), jnp.int32),
            pltpu.VMEM((_CH, 2 * D), jnp.int32),
            pltpu.VMEM((_CH, 2 * D), jnp.int32),
            pltpu.SemaphoreType.DMA,
            pltpu.SemaphoreType.DMA,
            pltpu.SemaphoreType.DMA,
            pltpu.SemaphoreType.DMA,
        ],
    )
    def k(ut_hbm, it_hbm, uid_hbm, iid_hbm, uout_hbm, iout_hbm,
          uidx_v, iidx_v, rows_a, rows_b, sem_a, sem_b, sem_sa, sem_sb):
        wid = lax.axis_index("s") * _NC + lax.axis_index("c")
        base = wid * _BPW
        sl = pl.ds(base, _BPW)
        pltpu.sync_copy(uid_hbm.at[sl], uidx_v)
        pltpu.sync_copy(iid_hbm.at[sl], iidx_v)

        @pl.loop(0, _BPW, step=16)
        def _(c):
            s = pl.ds(c, 16)
            u = uidx_v.at[s][...]
            uidx_v.at[s][...] = (
                lax.shift_left(lax.shift_right_logical(u, _PSH), _PSH - 2)
                + jnp.bitwise_and(u, _QPB - 1))
            v = iidx_v.at[s][...]
            iidx_v.at[s][...] = (
                lax.shift_left(lax.shift_right_logical(v, _PSH), _PSH - 2)
                + jnp.bitwise_and(v, _QPB - 1))

        work = []
        for tbl, idx_v, out_hbm in ((ut_hbm, uidx_v, uout_hbm),
                                    (it_hbm, iidx_v, iout_hbm)):
            for c in range(_BPW // _CH):
                work.append((tbl, idx_v.at[pl.ds(c * _CH, _CH)],
                             out_hbm.at[pl.ds(base + c * _CH, _CH)]))

        bufs = (rows_a, rows_b)
        gsems = (sem_a, sem_b)
        ssems = (sem_sa, sem_sb)
        n = len(work)
        gath = [None] * n
        stor = [None] * n
        for i in range(n):
            b = i % 2
            if i >= 2:
                stor[i - 2].wait()
            tbl, idx, out = work[i]
            gath[i] = pltpu.async_copy(tbl.at[idx], bufs[b], gsems[b])
            if i >= 1:
                gath[i - 1].wait()
                _, _, prev_out = work[i - 1]
                stor[i - 1] = pltpu.async_copy(bufs[(i - 1) % 2], prev_out,
                                               ssems[(i - 1) % 2])
        gath[n - 1].wait()
        stor[n - 1] = pltpu.async_copy(bufs[(n - 1) % 2], work[n - 1][2],
                                       ssems[(n - 1) % 2])
        stor[n - 2].wait()
        stor[n - 1].wait()

    return k(gu, gi, user_ids, item_ids)


_BLK = 2048


def _mlp_body(eu_ref, ei_ref, uids_ref, iids_ref,
              uw1, ub1, uw2, ub2, iw1, ib1, iw2, ib2,
              ou_ref, oi_ref):
    def tower(e2, ids, w1, b1, w2, b2):
        w = lax.bitcast_convert_type(e2, jnp.uint32)
        q_hi = jnp.bitwise_and(ids, 2 * _QPB) == 2 * _QPB
        bits = jnp.where(q_hi,
                         jnp.bitwise_and(w, jnp.uint32(0xFFFF0000)),
                         jnp.left_shift(w, jnp.uint32(16)))
        vals = lax.bitcast_convert_type(bits, jnp.float32)
        q_r = jnp.bitwise_and(ids, _QPB) == _QPB
        e = jnp.where(q_r, vals[:, D:], vals[:, :D])
        h = jnp.dot(e, w1, preferred_element_type=jnp.float32)
        h = jnp.maximum(h + b1, 0.0)
        o = jnp.dot(h, w2, preferred_element_type=jnp.float32)
        o = o + b2
        norm = jnp.sqrt(jnp.sum(o * o, axis=1, keepdims=True))
        return o / jnp.maximum(norm, 1e-12)

    ou_ref[...] = tower(eu_ref[...], uids_ref[...],
                        uw1[...], ub1[...], uw2[...], ub2[...])
    oi_ref[...] = tower(ei_ref[...], iids_ref[...],
                        iw1[...], ib1[...], iw2[...], ib2[...])


def _mlp_norm(e2_u, e2_i, uids, iids, uW1, ub1, uW2, ub2, iW1, ib1, iW2, ib2):
    blk2 = pl.BlockSpec((_BLK, 2 * D), lambda i: (i, 0))
    blk = pl.BlockSpec((_BLK, D), lambda i: (i, 0))
    ids_spec = pl.BlockSpec((_BLK, 1), lambda i: (i, 0))
    full = lambda shape: pl.BlockSpec(shape, lambda i: tuple(0 for _ in shape))
    return pl.pallas_call(
        _mlp_body,
        grid=(B // _BLK,),
        in_specs=[
            blk2, blk2, ids_spec, ids_spec,
            full((D, H)), full((1, H)), full((H, D)), full((1, D)),
            full((D, H)), full((1, H)), full((H, D)), full((1, D)),
        ],
        out_specs=[blk, blk],
        out_shape=(
            jax.ShapeDtypeStruct((B, D), jnp.float32),
            jax.ShapeDtypeStruct((B, D), jnp.float32),
        ),
    )(e2_u, e2_i, uids, iids, uW1, ub1, uW2, ub2, iW1, ib1, iW2, ib2)


def kernel(user_ids, item_ids, user_table, item_table,
           uW1, ub1, uW2, ub2, iW1, ib1, iW2, ib2):
    eye = jnp.eye(4 * D, dtype=jnp.float32)
    gu, gi = _prep(user_table.T, item_table.T, eye)
    e2_u, e2_i = _sc_gather_both(gu, gi, user_ids, item_ids)
    u_vec, i_vec = _mlp_norm(
        e2_u, e2_i,
        user_ids.reshape(B, 1), item_ids.reshape(B, 1),
        uW1, ub1.reshape(1, H), uW2, ub2.reshape(1, D),
        iW1, ib1.reshape(1, H), iW2, ib2.reshape(1, D),
    )
    return (u_vec, i_vec)


# speedup vs baseline: 3.9567x; 1.0241x over previous
import functools

import jax
import jax.numpy as jnp
from jax import lax
from jax.experimental import pallas as pl
from jax.experimental.pallas import tpu as pltpu
from jax.experimental.pallas import tpu_sc as plsc

B = 16384
D = 64
H = 2 * D
V = 1000000

_NC = 2
_NS = 16
_NW = _NC * _NS
_BPW = B // _NW
_CH = 256

_PB = 32768
_QPB = _PB // 4
_PSH = _PB.bit_length() - 1
_NPB = -(-V // _PB)
_GV = _NPB * _QPB


def _prep_body(ut_ref, it_ref, ey_ref, gu_ref, gi_ref):
    ey = ey_ref[...]

    def tr(x):
        xp = jnp.concatenate(
            [x[:, i * _QPB:(i + 1) * _QPB] for i in range(4)], axis=0)
        return lax.dot_general(xp, ey, (((0,), (0,)), ((), ())),
                               preferred_element_type=jnp.float32)

    def pack(xt):
        lo = xt[:, :2 * D].astype(jnp.bfloat16)
        hi = xt[:, 2 * D:].astype(jnp.bfloat16)
        lo32 = lax.bitcast_convert_type(lo, jnp.uint16).astype(jnp.uint32)
        hi32 = lax.bitcast_convert_type(hi, jnp.uint16).astype(jnp.uint32)
        w = jnp.bitwise_or(lo32, jnp.left_shift(hi32, jnp.uint32(16)))
        return lax.bitcast_convert_type(w, jnp.int32)

    gu_ref[...] = pack(tr(ut_ref[...]))
    gi_ref[...] = pack(tr(it_ref[...]))


def _prep(user_table_t, item_table_t, eye):
    inb = pl.BlockSpec((D, _PB), lambda i: (0, i))
    outb = pl.BlockSpec((_QPB, 2 * D), lambda i: (i, 0))
    return pl.pallas_call(
        _prep_body,
        grid=(pl.cdiv(V, _PB),),
        in_specs=[inb, inb, pl.BlockSpec((4 * D, 4 * D), lambda i: (0, 0))],
        out_specs=[outb, outb],
        out_shape=(
            jax.ShapeDtypeStruct((_GV, 2 * D), jnp.int32),
            jax.ShapeDtypeStruct((_GV, 2 * D), jnp.int32),
        ),
        compiler_params=pltpu.CompilerParams(
            dimension_semantics=("parallel",)),
    )(user_table_t, item_table_t, eye)


def _sc_gather_both(gu, gi, user_ids, item_ids):
    mesh = plsc.VectorSubcoreMesh(core_axis_name="c", subcore_axis_name="s")

    @functools.partial(
        pl.kernel,
        mesh=mesh,
        out_type=(
            jax.ShapeDtypeStruct((B, 2 * D), jnp.int32),
            jax.ShapeDtypeStruct((B, 2 * D), jnp.int32),
        ),
        scratch_types=[
            pltpu.VMEM((_BPW,), jnp.int32),
            pltpu.VMEM((_BPW,), jnp.int32),
            pltpu.VMEM((_CH, 2 * D), jnp.int32),
            pltpu.VMEM((_CH, 2 * D), jnp.int32),
            pltpu.SemaphoreType.DMA,
            pltpu.SemaphoreType.DMA,
            pltpu.SemaphoreType.DMA,
            pltpu.SemaphoreType.DMA,
        ],
    )
    def k(ut_hbm, it_hbm, uid_hbm, iid_hbm, uout_hbm, iout_hbm,
          uidx_v, iidx_v, rows_a, rows_b, sem_a, sem_b, sem_sa, sem_sb):
        wid = lax.axis_index("s") * _NC + lax.axis_index("c")
        base = wid * _BPW
        sl = pl.ds(base, _BPW)
        pltpu.sync_copy(uid_hbm.at[sl], uidx_v)
        pltpu.sync_copy(iid_hbm.at[sl], iidx_v)

        @pl.loop(0, _BPW, step=16)
        def _(c):
            s = pl.ds(c, 16)
            u = uidx_v.at[s][...]
            uidx_v.at[s][...] = (
                lax.shift_left(lax.shift_right_logical(u, _PSH), _PSH - 2)
                + jnp.bitwise_and(u, _QPB - 1))
            v = iidx_v.at[s][...]
            iidx_v.at[s][...] = (
                lax.shift_left(lax.shift_right_logical(v, _PSH), _PSH - 2)
                + jnp.bitwise_and(v, _QPB - 1))

        work = []
        for tbl, idx_v, out_hbm in ((ut_hbm, uidx_v, uout_hbm),
                                    (it_hbm, iidx_v, iout_hbm)):
            for c in range(_BPW // _CH):
                work.append((tbl, idx_v.at[pl.ds(c * _CH, _CH)],
                             out_hbm.at[pl.ds(base + c * _CH, _CH)]))

        bufs = (rows_a, rows_b)
        gsems = (sem_a, sem_b)
        ssems = (sem_sa, sem_sb)
        n = len(work)
        gath = [None] * n
        stor = [None] * n
        for i in range(n):
            b = i % 2
            if i >= 2:
                stor[i - 2].wait()
            tbl, idx, out = work[i]
            gath[i] = pltpu.async_copy(tbl.at[idx], bufs[b], gsems[b])
            if i >= 1:
                gath[i - 1].wait()
                _, _, prev_out = work[i - 1]
                stor[i - 1] = pltpu.async_copy(bufs[(i - 1) % 2], prev_out,
                                               ssems[(i - 1) % 2])
        gath[n - 1].wait()
        stor[n - 1] = pltpu.async_copy(bufs[(n - 1) % 2], work[n - 1][2],
                                       ssems[(n - 1) % 2])
        stor[n - 2].wait()
        stor[n - 1].wait()

    return k(gu, gi, user_ids, item_ids)


_BLK = 2048


def _mlp_body(eu_ref, ei_ref, uids_ref, iids_ref,
              uw1, ub1, uw2, ub2, iw1, ib1, iw2, ib2,
              ou_ref, oi_ref):
    def tower(e2, ids, w1, b1, w2, b2):
        w = lax.bitcast_convert_type(e2, jnp.uint32)
        q_hi = jnp.bitwise_and(ids, 2 * _QPB) == 2 * _QPB
        bits = jnp.where(q_hi,
                         jnp.bitwise_and(w, jnp.uint32(0xFFFF0000)),
                         jnp.left_shift(w, jnp.uint32(16)))
        vals = lax.bitcast_convert_type(bits, jnp.float32)
        q_r = jnp.bitwise_and(ids, _QPB) == _QPB
        e = jnp.where(q_r, vals[:, D:], vals[:, :D])
        h = jnp.dot(e, w1, preferred_element_type=jnp.float32)
        h = jnp.maximum(h + b1, 0.0)
        o = jnp.dot(h, w2, preferred_element_type=jnp.float32)
        o = o + b2
        norm = jnp.sqrt(jnp.sum(o * o, axis=1, keepdims=True))
        return o / jnp.maximum(norm, 1e-12)

    ou_ref[...] = tower(eu_ref[...], uids_ref[...],
                        uw1[...], ub1[...], uw2[...], ub2[...])
    oi_ref[...] = tower(ei_ref[...], iids_ref[...],
                        iw1[...], ib1[...], iw2[...], ib2[...])


def _mlp_norm(e2_u, e2_i, uids, iids, uW1, ub1, uW2, ub2, iW1, ib1, iW2, ib2):
    blk2 = pl.BlockSpec((_BLK, 2 * D), lambda i: (i, 0))
    blk = pl.BlockSpec((_BLK, D), lambda i: (i, 0))
    ids_spec = pl.BlockSpec((_BLK, 1), lambda i: (i, 0))
    full = lambda shape: pl.BlockSpec(shape, lambda i: tuple(0 for _ in shape))
    return pl.pallas_call(
        _mlp_body,
        grid=(B // _BLK,),
        in_specs=[
            blk2, blk2, ids_spec, ids_spec,
            full((D, H)), full((1, H)), full((H, D)), full((1, D)),
            full((D, H)), full((1, H)), full((H, D)), full((1, D)),
        ],
        out_specs=[blk, blk],
        out_shape=(
            jax.ShapeDtypeStruct((B, D), jnp.float32),
            jax.ShapeDtypeStruct((B, D), jnp.float32),
        ),
    )(e2_u, e2_i, uids, iids, uW1, ub1, uW2, ub2, iW1, ib1, iW2, ib2)


def kernel(user_ids, item_ids, user_table, item_table,
           uW1, ub1, uW2, ub2, iW1, ib1, iW2, ib2):
    eye = jnp.eye(4 * D, dtype=jnp.float32)
    gu, gi = _prep(user_table.T, item_table.T, eye)
    e2_u, e2_i = _sc_gather_both(gu, gi, user_ids, item_ids)
    u_vec, i_vec = _mlp_norm(
        e2_u, e2_i,
        user_ids.reshape(B, 1), item_ids.reshape(B, 1),
        uW1, ub1.reshape(1, H), uW2, ub2.reshape(1, D),
        iW1, ib1.reshape(1, H), iW2, ib2.reshape(1, D),
    )
    return (u_vec, i_vec)
